# Initial kernel scaffold; baseline (speedup 1.0000x reference)
import jax
import jax.numpy as jnp
from jax.experimental import pallas as pl


def kernel(x_user, x_item, edge_index_ui, edge_index_iu, W1, b1, g1, be1, W2, b2, g2, be2, W3, b3, g3, be3):
    raise NotImplementedError("write your pallas kernel here")


# speedup vs baseline: 8.2236x; 8.2236x over previous
import functools

import jax
import jax.numpy as jnp
from jax import lax
from jax.experimental import pallas as pl
from jax.experimental.pallas import tpu as pltpu
from jax.experimental.pallas import tpu_sc as plsc

N_HALF = 10000
N = 2 * N_HALF
D = 128
E = 160000
NS = 16
NC = 2
EPT = E // NS
CHUNK = 128
NCHUNK = (EPT + CHUNK - 1) // CHUNK
EPT_PAD = NCHUNK * CHUNK
ZROWS = 632
ACC_ROWS = NS * ZROWS
FROWS = 624
FTAIL = N_HALF - NS * FROWS
DUMMY = N_HALF
BN_EPS_K = 1e-5
R_BLK = 2000
GRID = N // R_BLK

_sc_mesh = plsc.VectorSubcoreMesh(core_axis_name="c", subcore_axis_name="s",
                                  num_cores=NC)


@functools.partial(
    pl.kernel,
    mesh=_sc_mesh,
    out_type=jax.ShapeDtypeStruct((N, D), jnp.float32),
    scratch_types=[
        pltpu.VMEM((NCHUNK, CHUNK), jnp.int32),
        pltpu.VMEM((NCHUNK, CHUNK), jnp.int32),
        pltpu.VMEM((CHUNK, D), jnp.float32),
        pltpu.VMEM_SHARED((ACC_ROWS, D), jnp.float32),
        pltpu.SemaphoreType.DMA,
    ],
)
def _sc_scatter(y_hbm, srcp_hbm, dstp_hbm, zeros_hbm, agg_hbm,
                src_v, dst_v, buf, acc_sh, gsem):
    c = lax.axis_index("c")
    s = lax.axis_index("s")
    pltpu.sync_copy(zeros_hbm.at[pl.ds(0, ZROWS)],
                    acc_sh.at[pl.ds(s * ZROWS, ZROWS)])
    pltpu.sync_copy(srcp_hbm.at[c, s], src_v)
    pltpu.sync_copy(dstp_hbm.at[c, s], dst_v)
    plsc.subcore_barrier()

    def body(j, carry):
        pltpu.async_copy(y_hbm.at[src_v.at[j]], buf, gsem).wait()
        pltpu.sync_copy(buf, acc_sh.at[dst_v.at[j]], add=True)
        return carry

    lax.fori_loop(0, NCHUNK, body, 0, unroll=False)
    plsc.subcore_barrier()
    pltpu.sync_copy(acc_sh.at[pl.ds(s * FROWS, FROWS)],
                    agg_hbm.at[pl.ds(c * N_HALF + s * FROWS, FROWS)])

    @pl.when(s == NS - 1)
    def _flush_tail():
        pltpu.sync_copy(acc_sh.at[pl.ds(NS * FROWS, FTAIL)],
                        agg_hbm.at[pl.ds(c * N_HALF + NS * FROWS, FTAIL)])


@functools.partial(
    pl.kernel,
    mesh=_sc_mesh,
    out_type=jax.ShapeDtypeStruct((N, D), jnp.float32),
    scratch_types=[
        pltpu.VMEM((NCHUNK, CHUNK), jnp.int32),
        pltpu.VMEM((CHUNK, D), jnp.float32),
        pltpu.VMEM_SHARED((ACC_ROWS, D), jnp.float32),
    ],
)
def _sc_deg(dstp_hbm, zeros_hbm, ones_hbm, deg_hbm, dst_v, ones_v, acc_sh):
    c = lax.axis_index("c")
    s = lax.axis_index("s")
    pltpu.sync_copy(zeros_hbm.at[pl.ds(0, ZROWS)],
                    acc_sh.at[pl.ds(s * ZROWS, ZROWS)])
    pltpu.sync_copy(dstp_hbm.at[c, s], dst_v)
    pltpu.sync_copy(ones_hbm, ones_v)
    plsc.subcore_barrier()

    def body(j, carry):
        pltpu.sync_copy(ones_v, acc_sh.at[dst_v.at[j]], add=True)
        return carry

    lax.fori_loop(0, NCHUNK, body, 0, unroll=False)
    plsc.subcore_barrier()
    pltpu.sync_copy(acc_sh.at[pl.ds(s * FROWS, FROWS)],
                    deg_hbm.at[pl.ds(c * N_HALF + s * FROWS, FROWS)])

    @pl.when(s == NS - 1)
    def _flush_tail():
        pltpu.sync_copy(acc_sh.at[pl.ds(NS * FROWS, FTAIL)],
                        deg_hbm.at[pl.ds(c * N_HALF + NS * FROWS, FTAIL)])


def _tc_prep_body(deg_ref, x_ref, w_ref, dis_ref, y_ref):
    deg = deg_ref[:, 0:1]
    dis = jnp.where(deg > 0.0, lax.rsqrt(deg), 0.0)
    dis_ref[:] = dis
    y_ref[:] = jnp.dot(dis * x_ref[:], w_ref[:],
                       preferred_element_type=jnp.float32)


_tc_prep = pl.pallas_call(
    _tc_prep_body,
    grid=(GRID,),
    in_specs=[
        pl.BlockSpec((R_BLK, D), lambda i: (i, 0)),
        pl.BlockSpec((R_BLK, D), lambda i: (i, 0)),
        pl.BlockSpec((D, D), lambda i: (0, 0)),
    ],
    out_specs=[
        pl.BlockSpec((R_BLK, 1), lambda i: (i, 0)),
        pl.BlockSpec((R_BLK, D), lambda i: (i, 0)),
    ],
    out_shape=[
        jax.ShapeDtypeStruct((N, 1), jnp.float32),
        jax.ShapeDtypeStruct((N, D), jnp.float32),
    ],
)


def _tc_mid_body(agg_ref, dis_ref, ab_ref, w_ref, y_ref):
    dis = dis_ref[:]
    h = jnp.maximum(ab_ref[0:1, :] * (dis * agg_ref[:]) + ab_ref[1:2, :], 0.0)
    y_ref[:] = jnp.dot(dis * h, w_ref[:], preferred_element_type=jnp.float32)


_tc_mid = pl.pallas_call(
    _tc_mid_body,
    grid=(GRID,),
    in_specs=[
        pl.BlockSpec((R_BLK, D), lambda i: (i, 0)),
        pl.BlockSpec((R_BLK, 1), lambda i: (i, 0)),
        pl.BlockSpec((2, D), lambda i: (0, 0)),
        pl.BlockSpec((D, D), lambda i: (0, 0)),
    ],
    out_specs=pl.BlockSpec((R_BLK, D), lambda i: (i, 0)),
    out_shape=jax.ShapeDtypeStruct((N, D), jnp.float32),
)


def _tc_fin_body(agg_ref, dis_ref, ab_ref, o_ref):
    dis = dis_ref[:]
    o_ref[:] = jnp.maximum(
        ab_ref[0:1, :] * (dis * agg_ref[:]) + ab_ref[1:2, :], 0.0)


_tc_fin = pl.pallas_call(
    _tc_fin_body,
    grid=(GRID,),
    in_specs=[
        pl.BlockSpec((R_BLK, D), lambda i: (i, 0)),
        pl.BlockSpec((R_BLK, 1), lambda i: (i, 0)),
        pl.BlockSpec((2, D), lambda i: (0, 0)),
    ],
    out_specs=pl.BlockSpec((R_BLK, D), lambda i: (i, 0)),
    out_shape=jax.ShapeDtypeStruct((N, D), jnp.float32),
)


def _prep_edges(src, dst):
    src = src.reshape(NS, EPT)
    dst = dst.reshape(NS, EPT)
    pad = EPT_PAD - EPT
    src = jnp.pad(src, ((0, 0), (0, pad)))
    dst = jnp.pad(dst, ((0, 0), (0, pad)), constant_values=DUMMY)
    return src.reshape(NS, NCHUNK, CHUNK), dst.reshape(NS, NCHUNK, CHUNK)


def kernel(x_user, x_item, edge_index_ui, edge_index_iu,
           W1, b1, g1, be1, W2, b2, g2, be2, W3, b3, g3, be3):
    f32 = jnp.float32
    x = jnp.concatenate([x_user, x_item], axis=0)
    ei_ui = edge_index_ui.astype(jnp.int32)
    ei_iu = edge_index_iu.astype(jnp.int32)
    s0, d0 = _prep_edges(ei_iu[0] + N_HALF, ei_iu[1])
    s1, d1 = _prep_edges(ei_ui[0], ei_ui[1])
    srcp = jnp.stack([s0, s1])
    dstp = jnp.stack([d0, d1])
    zeros = jnp.zeros((ZROWS, D), f32)
    ones = jnp.ones((CHUNK, D), f32)

    inv_std = 1.0 / jnp.sqrt(jnp.asarray(1.0 + BN_EPS_K, f32))

    def ab(g, b, be):
        a = g * inv_std
        return jnp.stack([a, a * b + be])

    deg16 = _sc_deg(dstp, zeros, ones)
    dis, y = _tc_prep(deg16, x, W1)
    agg = _sc_scatter(y, srcp, dstp, zeros)
    y = _tc_mid(agg, dis, ab(g1, b1, be1), W2)
    agg = _sc_scatter(y, srcp, dstp, zeros)
    y = _tc_mid(agg, dis, ab(g2, b2, be2), W3)
    agg = _sc_scatter(y, srcp, dstp, zeros)
    out = _tc_fin(agg, dis, ab(g3, b3, be3))
    return (out[:N_HALF], out[N_HALF:])
